# baseline (speedup 1.0000x reference)
import functools

import jax
import jax.numpy as jnp
from jax import lax
from jax.experimental import pallas as pl
from jax.experimental.pallas import tpu as pltpu
from jax.experimental.pallas import tpu_sc as plsc

BATCH = 4
SEQ = 8192
VOCAB = 8192
DIM = 1024

NUM_CORES = 2
NUM_SUBCORES = 16
NUM_WORKERS = NUM_CORES * NUM_SUBCORES
TOTAL = BATCH * SEQ
PER_WORKER = TOTAL // NUM_WORKERS
CHUNK = 16
NBUF = 6
LAG = 3
NUM_CHUNKS = PER_WORKER // CHUNK

_mesh = plsc.VectorSubcoreMesh(core_axis_name="c", subcore_axis_name="s")


@functools.partial(
    pl.kernel,
    mesh=_mesh,
    out_type=jax.ShapeDtypeStruct((TOTAL, DIM), jnp.float32),
    scratch_types=[
        pltpu.VMEM((PER_WORKER,), jnp.int32),
        pltpu.VMEM((NBUF, CHUNK, DIM), jnp.float32),
        pltpu.SemaphoreType.DMA,
        pltpu.SemaphoreType.DMA,
        pltpu.SemaphoreType.DMA,
        pltpu.SemaphoreType.DMA,
        pltpu.SemaphoreType.DMA,
        pltpu.SemaphoreType.DMA,
        pltpu.SemaphoreType.DMA,
        pltpu.SemaphoreType.DMA,
        pltpu.SemaphoreType.DMA,
        pltpu.SemaphoreType.DMA,
        pltpu.SemaphoreType.DMA,
        pltpu.SemaphoreType.DMA,
    ],
)
def _gather_rows(idx_hbm, table_hbm, out_hbm, idx_v, ring, *sems):
    gsems = sems[:NBUF]
    ssems = sems[NBUF:]
    wid = lax.axis_index("s") * NUM_CORES + lax.axis_index("c")
    base = wid * PER_WORKER
    pltpu.sync_copy(idx_hbm.at[pl.ds(base, PER_WORKER)], idx_v)

    def gather(chunk_id, b):
        pltpu.async_copy(
            table_hbm.at[idx_v.at[pl.ds(chunk_id * CHUNK, CHUNK)]],
            ring.at[b],
            gsems[b],
        )

    def wait_gather(b):
        pltpu.make_async_copy(
            table_hbm.at[idx_v.at[pl.ds(0, CHUNK)]], ring.at[b], gsems[b]
        ).wait()

    def store(chunk_id, b):
        pltpu.async_copy(
            ring.at[b],
            out_hbm.at[pl.ds(base + chunk_id * CHUNK, CHUNK)],
            ssems[b],
        )

    def wait_store(b):
        pltpu.make_async_copy(
            ring.at[b], out_hbm.at[pl.ds(base, CHUNK)], ssems[b]
        ).wait()

    for c in range(LAG):
        gather(c, c)

    def body(c, b, do_wait_store, do_gather):
        wait_gather(b)
        store(c, b)
        if do_wait_store:
            wait_store((b + NBUF - LAG) % NBUF)
        if do_gather:
            gather(c + LAG, (b + LAG) % NBUF)

    for c in range(LAG):
        body(c, c, False, True)

    @pl.loop(1, (NUM_CHUNKS - LAG - 1) // NBUF)
    def _steady(g):
        c0 = g * NBUF - LAG
        for b0 in range(NBUF):
            b = (b0 + LAG) % NBUF
            body(c0 + b0, b, True, True)

    steady_end = ((NUM_CHUNKS - LAG - 1) // NBUF) * NBUF - LAG
    for c in range(steady_end, NUM_CHUNKS):
        body(c, c % NBUF, True, c + LAG < NUM_CHUNKS)

    for c in range(NUM_CHUNKS - LAG, NUM_CHUNKS):
        wait_store(c % NBUF)


def kernel(position_ids, position_embeddings):
    idx = position_ids.reshape(-1)
    out = _gather_rows(idx, position_embeddings)
    return out.reshape(BATCH, SEQ, DIM)
